# baseline (speedup 1.0000x reference)
import functools

import jax
import jax.numpy as jnp
from jax import lax
from jax.experimental import pallas as pl
from jax.experimental.pallas import tpu as pltpu
from jax.experimental.pallas import tpu_sc as plsc

_N = 500000
_M = 32
_KSUB = 256
_DSUB = 4
_D = _M * _DSUB
_NC = 2
_NS = 16
_NW = _NC * _NS
_R = 128
_NCHUNK = 122
_TAIL0 = _NW * _NCHUNK * _R + 2 * _R
_TAIL = _N - _TAIL0


def _decode_body(codes_hbm, tbl_hbm, tailc_hbm, out_hbm, tbl_v, tc_v,
                 ca, cb, oa, ob, sia, sib, soa, sob):
    wid = lax.axis_index("s") * _NC + lax.axis_index("c")
    pltpu.sync_copy(tbl_hbm, tbl_v)

    iota = lax.iota(jnp.int32, 16)
    base_row = wid * _NCHUNK * _R

    def start_in(k, cv, sem):
        row0 = pl.multiple_of(base_row + k * _R, _R)
        pltpu.async_copy(codes_hbm.at[:, pl.ds(row0, _R)], cv, sem)

    def wait_in(cv, sem):
        pltpu.make_async_copy(
            codes_hbm.at[:, pl.ds(0, _R)], cv, sem).wait()

    def start_out(k, ov, sem):
        row0 = pl.multiple_of(base_row + k * _R, _R)
        pltpu.async_copy(ov, out_hbm.at[pl.ds(row0, _R)], sem)

    def wait_out(ov, sem):
        pltpu.make_async_copy(
            ov, out_hbm.at[pl.ds(0, _R)], sem).wait()

    def compute(cv, ov, nrows):
        @plsc.parallel_loop(0, nrows, unroll=2)
        def row_body(r):
            rv = jnp.full((16,), r, jnp.int32)
            for h in range(2):
                ch = plsc.load_gather(cv, [iota + h * 16, rv])
                base4 = (ch << 2) + (iota << 10) + (h * 16384)
                sbase = (iota << 2) + h * 64
                for d in range(4):
                    val = plsc.load_gather(tbl_v, [base4 + d])
                    plsc.store_scatter(ov, [rv, sbase + d], val)

    start_in(0, ca, sia)
    wait_in(ca, sia)
    start_in(1, cb, sib)
    compute(ca, oa, _R)
    start_out(0, oa, soa)
    wait_in(cb, sib)
    start_in(2, ca, sia)
    compute(cb, ob, _R)
    start_out(1, ob, sob)

    def pair_body(g, _):
        k0 = 2 * g
        wait_in(ca, sia)
        start_in(k0 + 1, cb, sib)
        wait_out(oa, soa)
        compute(ca, oa, _R)
        start_out(k0, oa, soa)
        wait_in(cb, sib)
        start_in(k0 + 2, ca, sia)
        wait_out(ob, sob)
        compute(cb, ob, _R)
        start_out(k0 + 1, ob, sob)
        return 0

    lax.fori_loop(1, _NCHUNK // 2 - 1, pair_body, 0)

    wait_in(ca, sia)
    start_in(_NCHUNK - 1, cb, sib)
    wait_out(oa, soa)
    compute(ca, oa, _R)
    start_out(_NCHUNK - 2, oa, soa)
    wait_in(cb, sib)
    wait_out(ob, sob)
    compute(cb, ob, _R)
    start_out(_NCHUNK - 1, ob, sob)
    wait_out(oa, soa)
    wait_out(ob, sob)

    for w, r0 in ((0, _NW * _NCHUNK * _R), (1, _NW * _NCHUNK * _R + _R)):
        @pl.when(wid == w)
        def _leftover(r0=r0):
            pltpu.sync_copy(codes_hbm.at[:, pl.ds(r0, _R)], ca)
            compute(ca, oa, _R)
            pltpu.sync_copy(oa, out_hbm.at[pl.ds(r0, _R)])

    @pl.when(wid == 2)
    def _tail():
        pltpu.sync_copy(tailc_hbm, tc_v)

        @plsc.parallel_loop(0, _TAIL, unroll=2)
        def tail_row(r):
            rv = jnp.full((16,), r, jnp.int32)
            for h in range(2):
                ch = plsc.load_gather(
                    tc_v, [jnp.full((16,), r * _M + h * 16, jnp.int32)
                           + iota])
                base4 = (ch << 2) + (iota << 10) + (h * 16384)
                sbase = (iota << 2) + h * 64
                for d in range(4):
                    val = plsc.load_gather(tbl_v, [base4 + d])
                    plsc.store_scatter(oa, [rv, sbase + d], val)

        pltpu.sync_copy(oa.at[pl.ds(0, _TAIL)],
                        out_hbm.at[pl.ds(_TAIL0, _TAIL)])


_mesh = plsc.VectorSubcoreMesh(core_axis_name="c", subcore_axis_name="s")

_decode = functools.partial(
    pl.kernel,
    mesh=_mesh,
    compiler_params=pltpu.CompilerParams(
        use_tc_tiling_on_sc=True, needs_layout_passes=False),
    out_type=jax.ShapeDtypeStruct((_N, _D), jnp.float32),
    scratch_types=[
        pltpu.VMEM((_M * _KSUB * _DSUB,), jnp.float32),
        pltpu.VMEM((_TAIL * _M,), jnp.int32),
        pltpu.VMEM((_M, _R), jnp.int32),
        pltpu.VMEM((_M, _R), jnp.int32),
        pltpu.VMEM((_R, _D), jnp.float32),
        pltpu.VMEM((_R, _D), jnp.float32),
        pltpu.SemaphoreType.DMA,
        pltpu.SemaphoreType.DMA,
        pltpu.SemaphoreType.DMA,
        pltpu.SemaphoreType.DMA,
    ],
)(_decode_body)


@jax.jit
def kernel(codes, centroids):
    tailc = codes[_TAIL0:].reshape(-1)
    return _decode(codes.T, centroids.reshape(-1), tailc)


# speedup vs baseline: 1.0641x; 1.0641x over previous
import functools

import jax
import jax.numpy as jnp
from jax import lax
from jax.experimental import pallas as pl
from jax.experimental.pallas import tpu as pltpu
from jax.experimental.pallas import tpu_sc as plsc

_N = 500000
_M = 32
_KSUB = 256
_DSUB = 4
_D = _M * _DSUB
_NC = 2
_NS = 16
_NW = _NC * _NS
_RPW = 15624
_R = 168
_NCHUNK = _RPW // _R
_TAIL = _N - _RPW * _NW


def _decode_body(codes_hbm, tbl_hbm, out_hbm, tbl_v,
                 ca, cb, oa, ob, sia, sib, soa, sob):
    wid = lax.axis_index("s") * _NC + lax.axis_index("c")
    pltpu.sync_copy(tbl_hbm, tbl_v)

    iota = lax.iota(jnp.int32, 16)
    base_row = wid * _RPW

    def start_in(k, cv, sem):
        row0 = pl.multiple_of(base_row + k * _R, 8)
        pltpu.async_copy(codes_hbm.at[pl.ds(row0, _R)], cv, sem)

    def wait_in(cv, sem):
        pltpu.make_async_copy(
            codes_hbm.at[pl.ds(0, _R)], cv, sem).wait()

    def start_out(k, ov, sem):
        row0 = pl.multiple_of(base_row + k * _R, 8)
        pltpu.async_copy(ov, out_hbm.at[pl.ds(row0, _R)], sem)

    def wait_out(ov, sem):
        pltpu.make_async_copy(
            ov, out_hbm.at[pl.ds(0, _R)], sem).wait()

    def compute(cv, ov, nrows):
        @plsc.parallel_loop(0, nrows, unroll=2)
        def row_body(r):
            rv = jnp.full((16,), r, jnp.int32)
            for h in range(2):
                ch = plsc.load_gather(cv, [rv, iota + h * 16])
                base4 = (ch << 2) + (iota << 10) + (h * 16384)
                sbase = (iota << 2) + h * 64
                for d in range(4):
                    val = plsc.load_gather(tbl_v, [base4 + d])
                    plsc.store_scatter(ov, [rv, sbase + d], val)

    start_in(0, ca, sia)
    wait_in(ca, sia)
    start_in(1, cb, sib)
    compute(ca, oa, _R)
    start_out(0, oa, soa)
    wait_in(cb, sib)
    start_in(2, ca, sia)
    compute(cb, ob, _R)
    start_out(1, ob, sob)

    def pair_body(g, _):
        k0 = 2 * g
        wait_in(ca, sia)
        start_in(k0 + 1, cb, sib)
        wait_out(oa, soa)
        compute(ca, oa, _R)
        start_out(k0, oa, soa)
        wait_in(cb, sib)
        start_in(k0 + 2, ca, sia)
        wait_out(ob, sob)
        compute(cb, ob, _R)
        start_out(k0 + 1, ob, sob)
        return 0

    lax.fori_loop(1, _NCHUNK // 2, pair_body, 0)

    wait_in(ca, sia)
    wait_out(oa, soa)
    compute(ca, oa, _R)
    start_out(_NCHUNK - 1, oa, soa)
    wait_out(oa, soa)
    wait_out(ob, sob)

    @pl.when(wid == _NW - 1)
    def _tail():
        t0 = _RPW * _NW
        pltpu.sync_copy(codes_hbm.at[pl.ds(t0, _TAIL)],
                        ca.at[pl.ds(0, _TAIL)])
        compute(ca, oa, _TAIL)
        pltpu.sync_copy(oa.at[pl.ds(0, _TAIL)],
                        out_hbm.at[pl.ds(t0, _TAIL)])


_mesh = plsc.VectorSubcoreMesh(core_axis_name="c", subcore_axis_name="s")

_decode = functools.partial(
    pl.kernel,
    mesh=_mesh,
    compiler_params=pltpu.CompilerParams(
        use_tc_tiling_on_sc=True, needs_layout_passes=False),
    out_type=jax.ShapeDtypeStruct((_N, _D), jnp.float32),
    scratch_types=[
        pltpu.VMEM((_M * _KSUB * _DSUB,), jnp.float32),
        pltpu.VMEM((_R, _M), jnp.int32),
        pltpu.VMEM((_R, _M), jnp.int32),
        pltpu.VMEM((_R, _D), jnp.float32),
        pltpu.VMEM((_R, _D), jnp.float32),
        pltpu.SemaphoreType.DMA,
        pltpu.SemaphoreType.DMA,
        pltpu.SemaphoreType.DMA,
        pltpu.SemaphoreType.DMA,
    ],
)(_decode_body)


@jax.jit
def kernel(codes, centroids):
    return _decode(codes, centroids.reshape(-1))
